# Initial kernel scaffold; baseline (speedup 1.0000x reference)
import jax
import jax.numpy as jnp
from jax.experimental import pallas as pl


def kernel(pos, W1, b1, W2, b2, W3, b3, pw, pb, m1w, m1b, m2w, m2b, m3w, m3b):
    raise NotImplementedError("write your pallas kernel here")


# speedup vs baseline: 2.3297x; 2.3297x over previous
import functools

import jax
import jax.numpy as jnp
from jax.experimental import pallas as pl

_HIGHEST = jax.lax.Precision.HIGHEST


def _dot(a, b):
    return jax.lax.dot_general(a, b, (((1,), (0,)), ((), ())),
                               precision=_HIGHEST,
                               preferred_element_type=jnp.float32)


def _dotd(a, b):
    return jax.lax.dot_general(a, b, (((1,), (0,)), ((), ())),
                               precision=jax.lax.Precision.DEFAULT,
                               preferred_element_type=jnp.float32)


def _knn_edgeconv_kernel(pos_blk_ref, pos_all_ref, posT_ref,
                         W1_ref, b1_ref, W2_ref, b2_ref, W3_ref, b3_ref,
                         pw_ref, pb_ref, out_ref, *, R, N, K):
    q = pos_blk_ref[0]
    p = pos_all_ref[0]

    d = jnp.zeros((R, N), dtype=jnp.float32)
    for c in range(3):
        qc = q[:, c:c + 1]
        pc = posT_ref[0, c, :].reshape(1, N)
        diff = qc - pc
        d = d + diff * diff

    col = jax.lax.broadcasted_iota(jnp.int32, (R, N), 1)

    W1 = W1_ref[...]
    b1 = b1_ref[...]
    W2 = W2_ref[...]
    b2 = b2_ref[...]
    W3 = W3_ref[...]
    b3 = b3_ref[...]

    fmax = None
    for k in range(K + 1):
        m = jnp.min(d, axis=1, keepdims=True)
        cand = jnp.where(d == m, col, N)
        idx = jnp.min(cand, axis=1, keepdims=True)
        sel = col == idx
        d = jnp.where(sel, jnp.inf, d)
        if k == 0:
            continue
        nb = _dot(sel.astype(jnp.float32), p)
        e = jnp.concatenate([q, nb, nb - q], axis=1)
        y1r = jax.nn.relu(_dotd(e, W1) + b1)
        y1 = jnp.concatenate([y1r, q], axis=1)
        y2r = jax.nn.relu(_dotd(y1, W2) + b2)
        y2 = jnp.concatenate([y2r, y1], axis=1)
        l3 = _dotd(y2, W3) + b3
        y3 = jnp.concatenate([l3, y2], axis=1)
        fmax = y3 if fmax is None else jnp.maximum(fmax, y3)

    pw = pw_ref[...]
    norm = jnp.sqrt(jnp.sum(pw * pw))
    score = (_dotd(fmax, pw) + pb_ref[0, 0]) / norm
    out_ref[0] = jnp.concatenate([fmax, score], axis=1)


def _gpool_adjust_kernel(feat_ref, pos_ref, srow_ref, scol_ref,
                         m1w_ref, m1b_ref, m2w_ref, m2b_ref,
                         m3w_ref, m3b_ref, out_ref, *, N, TOPK, CH):
    s_row = srow_ref[0]
    irow = jax.lax.broadcasted_iota(jnp.int32, (1, N), 1)

    rank = jnp.zeros((1, N), dtype=jnp.int32)
    for c in range(N // CH):
        s_col = scol_ref[0, c * CH:(c + 1) * CH, :]
        jcol = (jax.lax.broadcasted_iota(jnp.int32, (CH, 1), 0)
                + c * CH)
        gt = s_col > s_row
        eq = (s_col == s_row) & (jcol < irow)
        cnt = (gt | eq).astype(jnp.int32)
        rank = rank + jnp.sum(cnt, axis=0, keepdims=True)

    pos = pos_ref[0]
    feat = feat_ref[0, :, :99]
    s_colv = scol_ref[0]

    m1w = m1w_ref[...]
    m1b = m1b_ref[...]
    m2w = m2w_ref[...]
    m2b = m2b_ref[...]
    m3w = m3w_ref[...]
    m3b = m3b_ref[...]

    for r0 in range(0, TOPK, CH):
        rr = (jax.lax.broadcasted_iota(jnp.int32, (CH, 1), 0) + r0)
        P = (rank == rr).astype(jnp.float32)
        pos_sel = _dot(P, pos)
        feat_sel = _dot(P, feat)
        s_sel = _dot(P, s_colv)
        gate = jax.nn.sigmoid(s_sel)
        x_ds = feat_sel * gate
        h = jax.nn.relu(_dotd(x_ds, m1w) + m1b)
        h = jax.nn.relu(_dotd(h, m2w) + m2b)
        adj = _dotd(h, m3w) + m3b
        out_ref[0, r0:r0 + CH, :] = pos_sel + adj


def kernel(pos, W1, b1, W2, b2, W3, b3, pw, pb, m1w, m1b, m2w, m2b, m3w, m3b):
    B, N, _ = pos.shape
    K = 16
    R = 256
    TOPK = N // 2
    CH = 256

    posT = jnp.transpose(pos, (0, 2, 1))

    feat100 = pl.pallas_call(
        functools.partial(_knn_edgeconv_kernel, R=R, N=N, K=K),
        grid=(B, N // R),
        in_specs=[
            pl.BlockSpec((1, R, 3), lambda b, n: (b, n, 0)),
            pl.BlockSpec((1, N, 3), lambda b, n: (b, 0, 0)),
            pl.BlockSpec((1, 3, N), lambda b, n: (b, 0, 0)),
            pl.BlockSpec((9, 32), lambda b, n: (0, 0)),
            pl.BlockSpec((1, 32), lambda b, n: (0, 0)),
            pl.BlockSpec((35, 32), lambda b, n: (0, 0)),
            pl.BlockSpec((1, 32), lambda b, n: (0, 0)),
            pl.BlockSpec((67, 32), lambda b, n: (0, 0)),
            pl.BlockSpec((1, 32), lambda b, n: (0, 0)),
            pl.BlockSpec((99, 1), lambda b, n: (0, 0)),
            pl.BlockSpec((1, 1), lambda b, n: (0, 0)),
        ],
        out_specs=pl.BlockSpec((1, R, 100), lambda b, n: (b, n, 0)),
        out_shape=jax.ShapeDtypeStruct((B, N, 100), jnp.float32),
    )(pos, pos, posT,
      W1, b1.reshape(1, 32), W2, b2.reshape(1, 32), W3, b3.reshape(1, 32),
      pw, pb.reshape(1, 1))

    score = feat100[:, :, 99]
    s_row = score.reshape(B, 1, N)
    s_col = score.reshape(B, N, 1)

    out = pl.pallas_call(
        functools.partial(_gpool_adjust_kernel, N=N, TOPK=TOPK, CH=CH),
        grid=(B,),
        in_specs=[
            pl.BlockSpec((1, N, 100), lambda b: (b, 0, 0)),
            pl.BlockSpec((1, N, 3), lambda b: (b, 0, 0)),
            pl.BlockSpec((1, 1, N), lambda b: (b, 0, 0)),
            pl.BlockSpec((1, N, 1), lambda b: (b, 0, 0)),
            pl.BlockSpec((99, 49), lambda b: (0, 0)),
            pl.BlockSpec((1, 49), lambda b: (0, 0)),
            pl.BlockSpec((49, 24), lambda b: (0, 0)),
            pl.BlockSpec((1, 24), lambda b: (0, 0)),
            pl.BlockSpec((24, 3), lambda b: (0, 0)),
            pl.BlockSpec((1, 3), lambda b: (0, 0)),
        ],
        out_specs=pl.BlockSpec((1, TOPK, 3), lambda b: (b, 0, 0)),
        out_shape=jax.ShapeDtypeStruct((B, TOPK, 3), jnp.float32),
    )(feat100, pos, s_row, s_col,
      m1w, m1b.reshape(1, 49), m2w, m2b.reshape(1, 24), m3w, m3b.reshape(1, 3))

    return out


# speedup vs baseline: 5.6447x; 2.4229x over previous
import functools

import jax
import jax.numpy as jnp
from jax.experimental import pallas as pl
from jax.experimental.pallas import tpu as pltpu

_HIGHEST = jax.lax.Precision.HIGHEST


def _dot(a, b):
    return jax.lax.dot_general(a, b, (((1,), (0,)), ((), ())),
                               precision=_HIGHEST,
                               preferred_element_type=jnp.float32)


def _dotd(a, b):
    return jax.lax.dot_general(a, b, (((1,), (0,)), ((), ())),
                               precision=jax.lax.Precision.DEFAULT,
                               preferred_element_type=jnp.float32)


def _knn_edgeconv_kernel(pos_blk_ref, pos_all_ref, posT_ref,
                         W1_ref, b1_ref, W2_ref, b2_ref, W3_ref, b3_ref,
                         pw_ref, pb_ref, out_ref, e_scr, *, R, N, K):
    q = pos_blk_ref[0]
    p = pos_all_ref[0]

    d = jnp.zeros((R, N), dtype=jnp.float32)
    for c in range(3):
        qc = q[:, c:c + 1]
        pc = posT_ref[0, c, :].reshape(1, N)
        diff = qc - pc
        d = d + diff * diff

    col = jax.lax.broadcasted_iota(jnp.int32, (R, N), 1)

    W1 = W1_ref[...]
    b1 = b1_ref[...]
    W2 = W2_ref[...]
    b2 = b2_ref[...]
    W3 = W3_ref[...]
    b3 = b3_ref[...]

    pcs = [posT_ref[0, c, :].reshape(1, N) for c in range(3)]
    for k in range(K + 1):
        m = jnp.min(d, axis=1, keepdims=True)
        cand = jnp.where(d == m, col, N)
        idx = jnp.min(cand, axis=1, keepdims=True)
        sel = col == idx
        d = jnp.where(sel, jnp.inf, d)
        if k == 0:
            continue
        nb = jnp.concatenate(
            [jnp.min(jnp.where(sel, pc, jnp.inf), axis=1, keepdims=True)
             for pc in pcs], axis=1)
        e_scr[(k - 1) * R:k * R, :] = jnp.concatenate([q, nb, nb - q], axis=1)

    e = e_scr[...]
    qt = e[:, 0:3]
    y1r = jax.nn.relu(_dotd(e, W1) + b1)
    y1 = jnp.concatenate([y1r, qt], axis=1)
    y2r = jax.nn.relu(_dotd(y1, W2) + b2)
    y2 = jnp.concatenate([y2r, y1], axis=1)
    l3 = _dotd(y2, W3) + b3
    y3 = jnp.concatenate([l3, y2], axis=1)
    fmax = jnp.max(y3.reshape(K, R, 99), axis=0)

    pw = pw_ref[...]
    norm = jnp.sqrt(jnp.sum(pw * pw))
    score = (_dotd(fmax, pw) + pb_ref[0, 0]) / norm
    out_ref[0] = jnp.concatenate([fmax, score], axis=1)


def _gpool_adjust_kernel(feat_ref, pos_ref, srow_ref, scol_ref,
                         m1w_ref, m1b_ref, m2w_ref, m2b_ref,
                         m3w_ref, m3b_ref, out_ref, *, N, TOPK, CH):
    s_row = srow_ref[0]
    irow = jax.lax.broadcasted_iota(jnp.int32, (1, N), 1)

    rank = jnp.zeros((1, N), dtype=jnp.int32)
    for c in range(N // CH):
        s_col = scol_ref[0, c * CH:(c + 1) * CH, :]
        jcol = (jax.lax.broadcasted_iota(jnp.int32, (CH, 1), 0)
                + c * CH)
        gt = s_col > s_row
        eq = (s_col == s_row) & (jcol < irow)
        cnt = (gt | eq).astype(jnp.int32)
        rank = rank + jnp.sum(cnt, axis=0, keepdims=True)

    pos = pos_ref[0]
    feat = feat_ref[0, :, :99]
    s_colv = scol_ref[0]

    m1w = m1w_ref[...]
    m1b = m1b_ref[...]
    m2w = m2w_ref[...]
    m2b = m2b_ref[...]
    m3w = m3w_ref[...]
    m3b = m3b_ref[...]

    for r0 in range(0, TOPK, CH):
        rr = (jax.lax.broadcasted_iota(jnp.int32, (CH, 1), 0) + r0)
        P = (rank == rr).astype(jnp.float32)
        pos_sel = _dot(P, pos)
        feat_sel = _dot(P, feat)
        s_sel = _dot(P, s_colv)
        gate = jax.nn.sigmoid(s_sel)
        x_ds = feat_sel * gate
        h = jax.nn.relu(_dotd(x_ds, m1w) + m1b)
        h = jax.nn.relu(_dotd(h, m2w) + m2b)
        adj = _dotd(h, m3w) + m3b
        out_ref[0, r0:r0 + CH, :] = pos_sel + adj


def kernel(pos, W1, b1, W2, b2, W3, b3, pw, pb, m1w, m1b, m2w, m2b, m3w, m3b):
    B, N, _ = pos.shape
    K = 16
    R = 256
    TOPK = N // 2
    CH = 256

    posT = jnp.transpose(pos, (0, 2, 1))

    feat100 = pl.pallas_call(
        functools.partial(_knn_edgeconv_kernel, R=R, N=N, K=K),
        grid=(B, N // R),
        in_specs=[
            pl.BlockSpec((1, R, 3), lambda b, n: (b, n, 0)),
            pl.BlockSpec((1, N, 3), lambda b, n: (b, 0, 0)),
            pl.BlockSpec((1, 3, N), lambda b, n: (b, 0, 0)),
            pl.BlockSpec((9, 32), lambda b, n: (0, 0)),
            pl.BlockSpec((1, 32), lambda b, n: (0, 0)),
            pl.BlockSpec((35, 32), lambda b, n: (0, 0)),
            pl.BlockSpec((1, 32), lambda b, n: (0, 0)),
            pl.BlockSpec((67, 32), lambda b, n: (0, 0)),
            pl.BlockSpec((1, 32), lambda b, n: (0, 0)),
            pl.BlockSpec((99, 1), lambda b, n: (0, 0)),
            pl.BlockSpec((1, 1), lambda b, n: (0, 0)),
        ],
        out_specs=pl.BlockSpec((1, R, 100), lambda b, n: (b, n, 0)),
        out_shape=jax.ShapeDtypeStruct((B, N, 100), jnp.float32),
        scratch_shapes=[pltpu.VMEM((K * R, 9), jnp.float32)],
    )(pos, pos, posT,
      W1, b1.reshape(1, 32), W2, b2.reshape(1, 32), W3, b3.reshape(1, 32),
      pw, pb.reshape(1, 1))

    score = feat100[:, :, 99]
    s_row = score.reshape(B, 1, N)
    s_col = score.reshape(B, N, 1)

    out = pl.pallas_call(
        functools.partial(_gpool_adjust_kernel, N=N, TOPK=TOPK, CH=CH),
        grid=(B,),
        in_specs=[
            pl.BlockSpec((1, N, 100), lambda b: (b, 0, 0)),
            pl.BlockSpec((1, N, 3), lambda b: (b, 0, 0)),
            pl.BlockSpec((1, 1, N), lambda b: (b, 0, 0)),
            pl.BlockSpec((1, N, 1), lambda b: (b, 0, 0)),
            pl.BlockSpec((99, 49), lambda b: (0, 0)),
            pl.BlockSpec((1, 49), lambda b: (0, 0)),
            pl.BlockSpec((49, 24), lambda b: (0, 0)),
            pl.BlockSpec((1, 24), lambda b: (0, 0)),
            pl.BlockSpec((24, 3), lambda b: (0, 0)),
            pl.BlockSpec((1, 3), lambda b: (0, 0)),
        ],
        out_specs=pl.BlockSpec((1, TOPK, 3), lambda b: (b, 0, 0)),
        out_shape=jax.ShapeDtypeStruct((B, TOPK, 3), jnp.float32),
    )(feat100, pos, s_row, s_col,
      m1w, m1b.reshape(1, 49), m2w, m2b.reshape(1, 24), m3w, m3b.reshape(1, 3))

    return out
